# baseline (speedup 1.0000x reference)
import functools

import jax
import jax.numpy as jnp
from jax import lax
from jax.experimental import pallas as pl
from jax.experimental.pallas import tpu as pltpu
from jax.experimental.pallas import tpu_sc as plsc

_NC = 2
_NS = 16
_NW = _NC * _NS
_CHUNK = 128
_D = 128


def _pad_up(v, m):
    return (v + m - 1) // m * m


@functools.lru_cache(maxsize=None)
def _make_deg_kernel(ep, nrows):
    nchunks_w = ep // _CHUNK // _NW
    rows_s = nrows // _NS
    full = rows_s // _CHUNK
    rem = rows_s % _CHUNK
    mesh = plsc.VectorSubcoreMesh(core_axis_name="c", subcore_axis_name="s")

    @functools.partial(
        pl.kernel,
        out_type=jax.ShapeDtypeStruct((_NC, nrows, 16), jnp.float32),
        mesh=mesh,
        scratch_types=[
            pltpu.VMEM((_CHUNK,), jnp.int32),
            pltpu.VMEM((_CHUNK, 16), jnp.float32),
            pltpu.VMEM((_CHUNK, 16), jnp.float32),
            pltpu.VMEM_SHARED((nrows, 16), jnp.float32),
            pltpu.SemaphoreType.DMA,
        ],
    )
    def deg_kernel(dst_hbm, out_hbm, idx_v, ones_v, zero_v, acc_sh, sem):
        cid = lax.axis_index("c")
        sid = lax.axis_index("s")
        wid = sid * _NC + cid
        base = sid * rows_s

        @pl.loop(0, _CHUNK)
        def _fill(i):
            ones_v[i, :] = jnp.ones((16,), jnp.float32)
            zero_v[i, :] = jnp.zeros((16,), jnp.float32)

        @pl.loop(0, full)
        def _zero(i):
            pltpu.sync_copy(zero_v, acc_sh.at[pl.ds(base + i * _CHUNK, _CHUNK)])

        if rem:
            pltpu.sync_copy(zero_v.at[pl.ds(0, rem)],
                            acc_sh.at[pl.ds(base + full * _CHUNK, rem)])
        plsc.subcore_barrier()

        cbase = wid * nchunks_w

        @pl.loop(0, nchunks_w)
        def _body(i):
            pltpu.sync_copy(dst_hbm.at[pl.ds((cbase + i) * _CHUNK, _CHUNK)], idx_v)
            pltpu.sync_copy(ones_v, acc_sh.at[idx_v], add=True)

        plsc.subcore_barrier()
        pltpu.sync_copy(acc_sh.at[pl.ds(base, rows_s)],
                        out_hbm.at[cid, pl.ds(base, rows_s)])

    return deg_kernel


_PIECE = 40


@functools.lru_cache(maxsize=None)
def _make_conv_kernel(ep, nrows, nch0):
    nchunks = ep // _CHUNK
    nch1 = nchunks - nch0
    ns0, ns1 = nch0 // _NS, nch1 // _NS
    assert ns0 % _PIECE == 0 and ns1 % _PIECE == 0
    rows_s = nrows // _NS
    full = rows_s // _CHUNK
    rem = rows_s % _CHUNK
    mesh = plsc.VectorSubcoreMesh(core_axis_name="c", subcore_axis_name="s")

    @functools.partial(
        pl.kernel,
        out_type=jax.ShapeDtypeStruct((_NC, nrows, _D), jnp.float32),
        mesh=mesh,
        scratch_types=[
            pltpu.VMEM((_PIECE, _CHUNK), jnp.int32),
            pltpu.VMEM((_PIECE, _CHUNK), jnp.int32),
            pltpu.VMEM((_CHUNK, _D), jnp.float32),
            pltpu.VMEM((_CHUNK, _D), jnp.float32),
            pltpu.VMEM_SHARED((nrows, _D), jnp.float32),
            pltpu.SemaphoreType.DMA,
            pltpu.SemaphoreType.DMA,
        ],
    )
    def conv_kernel(tab_hbm, src_hbm, dst_hbm, out_hbm,
                    srcs, dsts, rows_a, rows_b, acc_sh, sem_a, sem_b):
        cid = lax.axis_index("c")
        sid = lax.axis_index("s")
        base = sid * rows_s
        cbase = jnp.where(cid == 0, sid * ns0, nch0 + sid * ns1)
        npieces = jnp.where(cid == 0, ns0 // _PIECE, ns1 // _PIECE)

        @pl.loop(0, _CHUNK)
        def _fill(i):
            for q in range(_D // 16):
                rows_a[i, pl.ds(q * 16, 16)] = jnp.zeros((16,), jnp.float32)

        @pl.loop(0, full)
        def _zero(i):
            pltpu.sync_copy(rows_a, acc_sh.at[pl.ds(base + i * _CHUNK, _CHUNK)])

        if rem:
            pltpu.sync_copy(rows_a.at[pl.ds(0, rem)],
                            acc_sh.at[pl.ds(base + full * _CHUNK, rem)])
        plsc.subcore_barrier()

        @pl.loop(0, npieces)
        def _piece(h):
            pltpu.sync_copy(src_hbm.at[pl.ds(cbase + h * _PIECE, _PIECE)], srcs)
            pltpu.sync_copy(dst_hbm.at[pl.ds(cbase + h * _PIECE, _PIECE)], dsts)

            pltpu.async_copy(tab_hbm.at[srcs.at[0]], rows_a, sem_a)

            @pl.loop(0, _PIECE, step=2)
            def _body(i):
                pltpu.async_copy(tab_hbm.at[srcs.at[i + 1]], rows_b, sem_b)
                pltpu.make_async_copy(tab_hbm.at[srcs.at[i]], rows_a,
                                      sem_a).wait()
                pltpu.sync_copy(rows_a, acc_sh.at[dsts.at[i]], add=True)

                @pl.when(i + 2 < _PIECE)
                def _():
                    pltpu.async_copy(tab_hbm.at[srcs.at[i + 2]], rows_a, sem_a)

                pltpu.make_async_copy(tab_hbm.at[srcs.at[i + 1]], rows_b,
                                      sem_b).wait()
                pltpu.sync_copy(rows_b, acc_sh.at[dsts.at[i + 1]], add=True)

        plsc.subcore_barrier()
        pltpu.sync_copy(acc_sh.at[pl.ds(base, rows_s)],
                        out_hbm.at[cid, pl.ds(base, rows_s)])

    return conv_kernel


def _dinv_of(da_ref, db_ref):
    deg = da_ref[:, 0:1] + db_ref[:, 0:1] + 1.0
    return lax.rsqrt(jnp.maximum(deg, 1.0))


def _stage_a_body(x_ref, wp_ref, bp_ref, w1_ref, da_ref, db_ref, o_ref):
    dinv = _dinv_of(da_ref, db_ref)
    h0 = jnp.dot(x_ref[...], wp_ref[...],
                 preferred_element_type=jnp.float32) + bp_ref[...]
    xw1 = jnp.dot(h0, w1_ref[...], preferred_element_type=jnp.float32)
    o_ref[...] = xw1 * dinv


def _stage_b_body(a0_ref, a1_ref, xws_ref, da_ref, db_ref, b1_ref, w2_ref,
                  o_ref):
    dinv = _dinv_of(da_ref, db_ref)
    s = a0_ref[...] + a1_ref[...] + xws_ref[...]
    h1 = jnp.maximum(dinv * s + b1_ref[...], 0.0)
    o_ref[...] = jnp.dot(h1, w2_ref[...],
                         preferred_element_type=jnp.float32) * dinv


def _stage_c_body(a0_ref, a1_ref, xws_ref, da_ref, db_ref, b2_ref, wl_ref,
                  bl_ref, o_ref):
    dinv = _dinv_of(da_ref, db_ref)
    h2 = dinv * (a0_ref[...] + a1_ref[...] + xws_ref[...]) + b2_ref[...]
    nrm = jnp.sqrt(jnp.sum(h2 * h2, axis=-1, keepdims=True))
    h2n = h2 / jnp.maximum(nrm, 1e-12)
    logits = jnp.dot(h2n, wl_ref[...],
                     preferred_element_type=jnp.float32) + bl_ref[...]
    m = jnp.max(logits, axis=-1, keepdims=True)
    lse = m + jnp.log(jnp.sum(jnp.exp(logits - m), axis=-1, keepdims=True))
    o_ref[...] = logits - lse


def _row_spec(r, c):
    return pl.BlockSpec((r, c), lambda i: (i, 0))


def _rep_spec(r, c):
    return pl.BlockSpec((r, c), lambda i: (0, 0))


def kernel(x, edge_index, W_pre, b_pre, W1, b1, W2, b2, W_lin, b_lin):
    n, d = x.shape
    e = edge_index.shape[1]
    ncls = W_lin.shape[1]
    ep = _pad_up(e, _NW * _CHUNK * 8)
    nrows = _pad_up(n + 1, _NS * 8)

    padv = jnp.full((ep - e,), n, jnp.int32)
    src = jnp.concatenate([edge_index[0], padv])
    dst = jnp.concatenate([edge_index[1], padv])

    deg = _make_deg_kernel(ep, nrows)(dst)
    dega, degb = deg[0, :n, :], deg[1, :n, :]

    rblk = 1000
    grid = (n // rblk,)

    xws1 = pl.pallas_call(
        _stage_a_body,
        grid=grid,
        in_specs=[_row_spec(rblk, d), _rep_spec(d, d), _rep_spec(1, d),
                  _rep_spec(d, d), _row_spec(rblk, 16), _row_spec(rblk, 16)],
        out_specs=_row_spec(rblk, d),
        out_shape=jax.ShapeDtypeStruct((n, d), jnp.float32),
    )(x, W_pre, b_pre.reshape(1, d), W1, dega, degb)

    nch0 = ep // _CHUNK
    conv = _make_conv_kernel(ep, nrows, nch0)
    zrow = jnp.zeros((1, d), jnp.float32)
    src2 = src.reshape(ep // _CHUNK, _CHUNK)
    dst2 = dst.reshape(ep // _CHUNK, _CHUNK)

    acc1 = conv(jnp.concatenate([xws1, zrow], axis=0), src2, dst2)
    xws2 = pl.pallas_call(
        _stage_b_body,
        grid=grid,
        in_specs=[_row_spec(rblk, d), _row_spec(rblk, d), _row_spec(rblk, d),
                  _row_spec(rblk, 16), _row_spec(rblk, 16), _rep_spec(1, d),
                  _rep_spec(d, d)],
        out_specs=_row_spec(rblk, d),
        out_shape=jax.ShapeDtypeStruct((n, d), jnp.float32),
    )(acc1[0, :n], acc1[1, :n], xws1, dega, degb, b1.reshape(1, d), W2)

    acc2 = conv(jnp.concatenate([xws2, zrow], axis=0), src2, dst2)
    out = pl.pallas_call(
        _stage_c_body,
        grid=grid,
        in_specs=[_row_spec(rblk, d), _row_spec(rblk, d), _row_spec(rblk, d),
                  _row_spec(rblk, 16), _row_spec(rblk, 16), _rep_spec(1, d),
                  _rep_spec(d, ncls), _rep_spec(1, ncls)],
        out_specs=_row_spec(rblk, ncls),
        out_shape=jax.ShapeDtypeStruct((n, ncls), jnp.float32),
    )(acc2[0, :n], acc2[1, :n], xws2, dega, degb, b2.reshape(1, d),
      W_lin, b_lin.reshape(1, ncls))

    return out


# speedup vs baseline: 1.0566x; 1.0566x over previous
import functools

import jax
import jax.numpy as jnp
from jax import lax
from jax.experimental import pallas as pl
from jax.experimental.pallas import tpu as pltpu
from jax.experimental.pallas import tpu_sc as plsc

_NC = 2
_NS = 16
_NW = _NC * _NS
_CHUNK = 128
_D = 128


def _pad_up(v, m):
    return (v + m - 1) // m * m


@functools.lru_cache(maxsize=None)
def _make_deg_kernel(ep, nrows):
    nchunks_w = ep // _CHUNK // _NW
    rows_s = nrows // _NS
    full = rows_s // _CHUNK
    rem = rows_s % _CHUNK
    mesh = plsc.VectorSubcoreMesh(core_axis_name="c", subcore_axis_name="s")

    @functools.partial(
        pl.kernel,
        out_type=jax.ShapeDtypeStruct((_NC, nrows, 16), jnp.float32),
        mesh=mesh,
        scratch_types=[
            pltpu.VMEM((_CHUNK,), jnp.int32),
            pltpu.VMEM((_CHUNK, 16), jnp.float32),
            pltpu.VMEM((_CHUNK, 16), jnp.float32),
            pltpu.VMEM_SHARED((nrows, 16), jnp.float32),
            pltpu.SemaphoreType.DMA,
        ],
    )
    def deg_kernel(dst_hbm, out_hbm, idx_v, ones_v, zero_v, acc_sh, sem):
        cid = lax.axis_index("c")
        sid = lax.axis_index("s")
        wid = sid * _NC + cid
        base = sid * rows_s

        @pl.loop(0, _CHUNK)
        def _fill(i):
            ones_v[i, :] = jnp.ones((16,), jnp.float32)
            zero_v[i, :] = jnp.zeros((16,), jnp.float32)

        @pl.loop(0, full)
        def _zero(i):
            pltpu.sync_copy(zero_v, acc_sh.at[pl.ds(base + i * _CHUNK, _CHUNK)])

        if rem:
            pltpu.sync_copy(zero_v.at[pl.ds(0, rem)],
                            acc_sh.at[pl.ds(base + full * _CHUNK, rem)])
        plsc.subcore_barrier()

        cbase = wid * nchunks_w

        @pl.loop(0, nchunks_w)
        def _body(i):
            pltpu.sync_copy(dst_hbm.at[pl.ds((cbase + i) * _CHUNK, _CHUNK)], idx_v)
            pltpu.sync_copy(ones_v, acc_sh.at[idx_v], add=True)

        plsc.subcore_barrier()
        pltpu.sync_copy(acc_sh.at[pl.ds(base, rows_s)],
                        out_hbm.at[cid, pl.ds(base, rows_s)])

    return deg_kernel


_PIECE = 40


@functools.lru_cache(maxsize=None)
def _make_conv_kernel(ep, nrows):
    nchunks = ep // _CHUNK
    ns = nchunks // _NW
    assert ns % _PIECE == 0
    rows_s = nrows // _NS
    full = rows_s // _CHUNK
    rem = rows_s % _CHUNK
    mesh = plsc.VectorSubcoreMesh(core_axis_name="c", subcore_axis_name="s")

    @functools.partial(
        pl.kernel,
        out_type=jax.ShapeDtypeStruct((_NC, nrows, _D), jnp.float32),
        mesh=mesh,
        scratch_types=[
            pltpu.VMEM((_PIECE, _CHUNK), jnp.int32),
            pltpu.VMEM((_PIECE, _CHUNK), jnp.int32),
            pltpu.VMEM((_CHUNK, _D), jnp.float32),
            pltpu.VMEM_SHARED((nrows, _D), jnp.float32),
            pltpu.SemaphoreType.DMA,
        ],
    )
    def conv_kernel(tab_hbm, src_hbm, dst_hbm, out_hbm,
                    srcs, dsts, rows_v, acc_sh, sem):
        cid = lax.axis_index("c")
        sid = lax.axis_index("s")
        wid = sid * _NC + cid
        base = sid * rows_s
        cbase = wid * ns

        @pl.loop(0, _CHUNK)
        def _fill(i):
            for q in range(_D // 16):
                rows_v[i, pl.ds(q * 16, 16)] = jnp.zeros((16,), jnp.float32)

        @pl.loop(0, full)
        def _zero(i):
            pltpu.sync_copy(rows_v, acc_sh.at[pl.ds(base + i * _CHUNK, _CHUNK)])

        if rem:
            pltpu.sync_copy(rows_v.at[pl.ds(0, rem)],
                            acc_sh.at[pl.ds(base + full * _CHUNK, rem)])
        plsc.subcore_barrier()

        @pl.loop(0, ns // _PIECE)
        def _piece(h):
            pltpu.sync_copy(src_hbm.at[pl.ds(cbase + h * _PIECE, _PIECE)],
                            srcs)
            pltpu.sync_copy(dst_hbm.at[pl.ds(cbase + h * _PIECE, _PIECE)],
                            dsts)

            @pl.loop(0, _PIECE)
            def _body(i):
                pltpu.async_copy(tab_hbm.at[srcs.at[i]], rows_v, sem).wait()
                pltpu.sync_copy(rows_v, acc_sh.at[dsts.at[i]], add=True)

        plsc.subcore_barrier()
        pltpu.sync_copy(acc_sh.at[pl.ds(base, rows_s)],
                        out_hbm.at[cid, pl.ds(base, rows_s)])

    return conv_kernel


def _dinv_of(da_ref, db_ref):
    deg = da_ref[:, 0:1] + db_ref[:, 0:1] + 1.0
    return lax.rsqrt(jnp.maximum(deg, 1.0))


def _stage_a_body(x_ref, wp_ref, bp_ref, w1_ref, da_ref, db_ref, o_ref):
    dinv = _dinv_of(da_ref, db_ref)
    h0 = jnp.dot(x_ref[...], wp_ref[...],
                 preferred_element_type=jnp.float32) + bp_ref[...]
    xw1 = jnp.dot(h0, w1_ref[...], preferred_element_type=jnp.float32)
    o_ref[...] = xw1 * dinv


def _stage_b_body(a0_ref, a1_ref, xws_ref, da_ref, db_ref, b1_ref, w2_ref,
                  o_ref):
    dinv = _dinv_of(da_ref, db_ref)
    s = a0_ref[...] + a1_ref[...] + xws_ref[...]
    h1 = jnp.maximum(dinv * s + b1_ref[...], 0.0)
    o_ref[...] = jnp.dot(h1, w2_ref[...],
                         preferred_element_type=jnp.float32) * dinv


def _stage_c_body(a0_ref, a1_ref, xws_ref, da_ref, db_ref, b2_ref, wl_ref,
                  bl_ref, o_ref):
    dinv = _dinv_of(da_ref, db_ref)
    h2 = dinv * (a0_ref[...] + a1_ref[...] + xws_ref[...]) + b2_ref[...]
    nrm = jnp.sqrt(jnp.sum(h2 * h2, axis=-1, keepdims=True))
    h2n = h2 / jnp.maximum(nrm, 1e-12)
    logits = jnp.dot(h2n, wl_ref[...],
                     preferred_element_type=jnp.float32) + bl_ref[...]
    m = jnp.max(logits, axis=-1, keepdims=True)
    lse = m + jnp.log(jnp.sum(jnp.exp(logits - m), axis=-1, keepdims=True))
    o_ref[...] = logits - lse


def _row_spec(r, c):
    return pl.BlockSpec((r, c), lambda i: (i, 0))


def _rep_spec(r, c):
    return pl.BlockSpec((r, c), lambda i: (0, 0))


def kernel(x, edge_index, W_pre, b_pre, W1, b1, W2, b2, W_lin, b_lin):
    n, d = x.shape
    e = edge_index.shape[1]
    ncls = W_lin.shape[1]
    ep = _pad_up(e, _NW * _CHUNK * 8)
    nrows = _pad_up(n + 1, _NS * 8)

    padv = jnp.full((ep - e,), n, jnp.int32)
    src = jnp.concatenate([edge_index[0], padv])
    dst = jnp.concatenate([edge_index[1], padv])

    deg = _make_deg_kernel(ep, nrows)(dst)
    dega, degb = deg[0, :n, :], deg[1, :n, :]

    rblk = 1000
    grid = (n // rblk,)

    xws1 = pl.pallas_call(
        _stage_a_body,
        grid=grid,
        in_specs=[_row_spec(rblk, d), _rep_spec(d, d), _rep_spec(1, d),
                  _rep_spec(d, d), _row_spec(rblk, 16), _row_spec(rblk, 16)],
        out_specs=_row_spec(rblk, d),
        out_shape=jax.ShapeDtypeStruct((n, d), jnp.float32),
    )(x, W_pre, b_pre.reshape(1, d), W1, dega, degb)

    conv = _make_conv_kernel(ep, nrows)
    zrow = jnp.zeros((1, d), jnp.float32)
    src2 = src.reshape(ep // _CHUNK, _CHUNK)
    dst2 = dst.reshape(ep // _CHUNK, _CHUNK)

    acc1 = conv(jnp.concatenate([xws1, zrow], axis=0), src2, dst2)
    xws2 = pl.pallas_call(
        _stage_b_body,
        grid=grid,
        in_specs=[_row_spec(rblk, d), _row_spec(rblk, d), _row_spec(rblk, d),
                  _row_spec(rblk, 16), _row_spec(rblk, 16), _rep_spec(1, d),
                  _rep_spec(d, d)],
        out_specs=_row_spec(rblk, d),
        out_shape=jax.ShapeDtypeStruct((n, d), jnp.float32),
    )(acc1[0, :n], acc1[1, :n], xws1, dega, degb, b1.reshape(1, d), W2)

    acc2 = conv(jnp.concatenate([xws2, zrow], axis=0), src2, dst2)
    out = pl.pallas_call(
        _stage_c_body,
        grid=grid,
        in_specs=[_row_spec(rblk, d), _row_spec(rblk, d), _row_spec(rblk, d),
                  _row_spec(rblk, 16), _row_spec(rblk, 16), _rep_spec(1, d),
                  _rep_spec(d, ncls), _rep_spec(1, ncls)],
        out_specs=_row_spec(rblk, ncls),
        out_shape=jax.ShapeDtypeStruct((n, ncls), jnp.float32),
    )(acc2[0, :n], acc2[1, :n], xws2, dega, degb, b2.reshape(1, d),
      W_lin, b_lin.reshape(1, ncls))

    return out


# speedup vs baseline: 1.2593x; 1.1919x over previous
import functools

import jax
import jax.numpy as jnp
from jax import lax
from jax.experimental import pallas as pl
from jax.experimental.pallas import tpu as pltpu
from jax.experimental.pallas import tpu_sc as plsc

_NC = 2
_NS = 16
_NW = _NC * _NS
_CHUNK = 128
_D = 128


def _pad_up(v, m):
    return (v + m - 1) // m * m


@functools.lru_cache(maxsize=None)
def _make_deg_kernel(ep, nrows):
    nchunks_w = ep // _CHUNK // _NW
    rows_s = nrows // _NS
    full = rows_s // _CHUNK
    rem = rows_s % _CHUNK
    mesh = plsc.VectorSubcoreMesh(core_axis_name="c", subcore_axis_name="s")

    @functools.partial(
        pl.kernel,
        out_type=jax.ShapeDtypeStruct((_NC, nrows, 16), jnp.float32),
        mesh=mesh,
        scratch_types=[
            pltpu.VMEM((_CHUNK,), jnp.int32),
            pltpu.VMEM((_CHUNK, 16), jnp.float32),
            pltpu.VMEM((_CHUNK, 16), jnp.float32),
            pltpu.VMEM_SHARED((nrows, 16), jnp.float32),
            pltpu.SemaphoreType.DMA,
        ],
    )
    def deg_kernel(dst_hbm, out_hbm, idx_v, ones_v, zero_v, acc_sh, sem):
        cid = lax.axis_index("c")
        sid = lax.axis_index("s")
        wid = sid * _NC + cid
        base = sid * rows_s

        @pl.loop(0, _CHUNK)
        def _fill(i):
            ones_v[i, :] = jnp.ones((16,), jnp.float32)
            zero_v[i, :] = jnp.zeros((16,), jnp.float32)

        @pl.loop(0, full)
        def _zero(i):
            pltpu.sync_copy(zero_v, acc_sh.at[pl.ds(base + i * _CHUNK, _CHUNK)])

        if rem:
            pltpu.sync_copy(zero_v.at[pl.ds(0, rem)],
                            acc_sh.at[pl.ds(base + full * _CHUNK, rem)])
        plsc.subcore_barrier()

        cbase = wid * nchunks_w

        @pl.loop(0, nchunks_w)
        def _body(i):
            pltpu.sync_copy(dst_hbm.at[pl.ds((cbase + i) * _CHUNK, _CHUNK)], idx_v)
            pltpu.sync_copy(ones_v, acc_sh.at[idx_v], add=True)

        plsc.subcore_barrier()
        pltpu.sync_copy(acc_sh.at[pl.ds(base, rows_s)],
                        out_hbm.at[cid, pl.ds(base, rows_s)])

    return deg_kernel


@functools.lru_cache(maxsize=None)
def _make_conv_kernel(ep, nrows):
    nchunks_w = ep // _CHUNK // _NW
    rows_s = nrows // _NS
    full = rows_s // _CHUNK
    rem = rows_s % _CHUNK
    mesh = plsc.VectorSubcoreMesh(core_axis_name="c", subcore_axis_name="s")

    @functools.partial(
        pl.kernel,
        out_type=jax.ShapeDtypeStruct((_NC, nrows, _D), jnp.float32),
        mesh=mesh,
        scratch_types=[
            pltpu.VMEM((_CHUNK,), jnp.int32),
            pltpu.VMEM((_CHUNK,), jnp.int32),
            pltpu.VMEM((_CHUNK, _D), jnp.float32),
            pltpu.VMEM_SHARED((nrows, _D), jnp.float32),
            pltpu.SemaphoreType.DMA,
        ],
    )
    def conv_kernel(tab_hbm, src_hbm, dst_hbm, out_hbm,
                    srcv, dstv, rows_v, acc_sh, sem):
        cid = lax.axis_index("c")
        sid = lax.axis_index("s")
        wid = sid * _NC + cid
        base = sid * rows_s

        @pl.loop(0, _CHUNK)
        def _fill(i):
            for q in range(_D // 16):
                rows_v[i, pl.ds(q * 16, 16)] = jnp.zeros((16,), jnp.float32)

        @pl.loop(0, full)
        def _zero(i):
            pltpu.sync_copy(rows_v, acc_sh.at[pl.ds(base + i * _CHUNK, _CHUNK)])

        if rem:
            pltpu.sync_copy(rows_v.at[pl.ds(0, rem)],
                            acc_sh.at[pl.ds(base + full * _CHUNK, rem)])
        plsc.subcore_barrier()

        cbase = wid * nchunks_w

        @pl.loop(0, nchunks_w)
        def _body(i):
            off = (cbase + i) * _CHUNK
            pltpu.sync_copy(src_hbm.at[pl.ds(off, _CHUNK)], srcv)
            pltpu.sync_copy(dst_hbm.at[pl.ds(off, _CHUNK)], dstv)
            pltpu.async_copy(tab_hbm.at[srcv], rows_v, sem).wait()
            pltpu.sync_copy(rows_v, acc_sh.at[dstv], add=True)

        plsc.subcore_barrier()
        pltpu.sync_copy(acc_sh.at[pl.ds(base, rows_s)],
                        out_hbm.at[cid, pl.ds(base, rows_s)])

    return conv_kernel


def _dinv_of(da_ref, db_ref):
    deg = da_ref[:, 0:1] + db_ref[:, 0:1] + 1.0
    return lax.rsqrt(jnp.maximum(deg, 1.0))


def _stage_a_body(x_ref, wp_ref, bp_ref, w1_ref, da_ref, db_ref, o_ref):
    dinv = _dinv_of(da_ref, db_ref)
    h0 = jnp.dot(x_ref[...], wp_ref[...],
                 preferred_element_type=jnp.float32) + bp_ref[...]
    xw1 = jnp.dot(h0, w1_ref[...], preferred_element_type=jnp.float32)
    o_ref[...] = xw1 * dinv


def _stage_b_body(a0_ref, a1_ref, xws_ref, da_ref, db_ref, b1_ref, w2_ref,
                  o_ref):
    dinv = _dinv_of(da_ref, db_ref)
    s = a0_ref[...] + a1_ref[...] + xws_ref[...]
    h1 = jnp.maximum(dinv * s + b1_ref[...], 0.0)
    o_ref[...] = jnp.dot(h1, w2_ref[...],
                         preferred_element_type=jnp.float32) * dinv


def _stage_c_body(a0_ref, a1_ref, xws_ref, da_ref, db_ref, b2_ref, wl_ref,
                  bl_ref, o_ref):
    dinv = _dinv_of(da_ref, db_ref)
    h2 = dinv * (a0_ref[...] + a1_ref[...] + xws_ref[...]) + b2_ref[...]
    nrm = jnp.sqrt(jnp.sum(h2 * h2, axis=-1, keepdims=True))
    h2n = h2 / jnp.maximum(nrm, 1e-12)
    logits = jnp.dot(h2n, wl_ref[...],
                     preferred_element_type=jnp.float32) + bl_ref[...]
    m = jnp.max(logits, axis=-1, keepdims=True)
    lse = m + jnp.log(jnp.sum(jnp.exp(logits - m), axis=-1, keepdims=True))
    o_ref[...] = logits - lse


def _row_spec(r, c):
    return pl.BlockSpec((r, c), lambda i: (i, 0))


def _rep_spec(r, c):
    return pl.BlockSpec((r, c), lambda i: (0, 0))


def kernel(x, edge_index, W_pre, b_pre, W1, b1, W2, b2, W_lin, b_lin):
    n, d = x.shape
    e = edge_index.shape[1]
    ncls = W_lin.shape[1]
    ep = _pad_up(e, _NW * _CHUNK)
    nrows = _pad_up(n + 1, _NS * 8)

    padv = jnp.full((ep - e,), n, jnp.int32)
    src = jnp.concatenate([edge_index[0], padv])
    dst = jnp.concatenate([edge_index[1], padv])

    deg = _make_deg_kernel(ep, nrows)(dst)
    dega, degb = deg[0, :n, :], deg[1, :n, :]

    rblk = 1000
    grid = (n // rblk,)

    xws1 = pl.pallas_call(
        _stage_a_body,
        grid=grid,
        in_specs=[_row_spec(rblk, d), _rep_spec(d, d), _rep_spec(1, d),
                  _rep_spec(d, d), _row_spec(rblk, 16), _row_spec(rblk, 16)],
        out_specs=_row_spec(rblk, d),
        out_shape=jax.ShapeDtypeStruct((n, d), jnp.float32),
    )(x, W_pre, b_pre.reshape(1, d), W1, dega, degb)

    conv = _make_conv_kernel(ep, nrows)
    zrow = jnp.zeros((1, d), jnp.float32)

    acc1 = conv(jnp.concatenate([xws1, zrow], axis=0), src, dst)
    xws2 = pl.pallas_call(
        _stage_b_body,
        grid=grid,
        in_specs=[_row_spec(rblk, d), _row_spec(rblk, d), _row_spec(rblk, d),
                  _row_spec(rblk, 16), _row_spec(rblk, 16), _rep_spec(1, d),
                  _rep_spec(d, d)],
        out_specs=_row_spec(rblk, d),
        out_shape=jax.ShapeDtypeStruct((n, d), jnp.float32),
    )(acc1[0, :n], acc1[1, :n], xws1, dega, degb, b1.reshape(1, d), W2)

    acc2 = conv(jnp.concatenate([xws2, zrow], axis=0), src, dst)
    out = pl.pallas_call(
        _stage_c_body,
        grid=grid,
        in_specs=[_row_spec(rblk, d), _row_spec(rblk, d), _row_spec(rblk, d),
                  _row_spec(rblk, 16), _row_spec(rblk, 16), _rep_spec(1, d),
                  _rep_spec(d, ncls), _rep_spec(1, ncls)],
        out_specs=_row_spec(rblk, ncls),
        out_shape=jax.ShapeDtypeStruct((n, ncls), jnp.float32),
    )(acc2[0, :n], acc2[1, :n], xws2, dega, degb, b2.reshape(1, d),
      W_lin, b_lin.reshape(1, ncls))

    return out
